# baseline (speedup 1.0000x reference)
import functools
import jax
import jax.numpy as jnp
from jax import lax
from jax.experimental import pallas as pl
from jax.experimental.pallas import tpu as pltpu
from jax.experimental.pallas import tpu_sc as plsc

_NUM_NODES = 19
_NUM_BANDS = 5
_BATCH = 4096
_N = _BATCH * _NUM_NODES
_E = 1245184
_EPS = 1e-5
_NC, _NS = 2, 16
_NW = _NC * _NS
_NPT = _N // _NS
_CH = 2048
_CHR = _CH // 128
_EPW = _E // _NW
_NCHUNK = _EPW // _CH
_ROWS_PW = _EPW // 128


def _sc_mesh():
  return plsc.VectorSubcoreMesh(core_axis_name="c", subcore_axis_name="s",
                                num_cores=_NC, num_subcores=_NS)


_SC_PARAMS = pltpu.CompilerParams(needs_layout_passes=False,
                                 use_tc_tiling_on_sc=False)


def _deg_body(cf_h, ew_h, cs_h, out_h, shf, shs, idxb, updb, onesb, zrow, sem):
  ci = lax.axis_index("c")
  si = lax.axis_index("s")
  wid = si * _NC + ci
  for t in range(8):
    onesb[0, pl.ds(t * 16, 16)] = jnp.full((16,), 1.0, jnp.float32)
    zrow[0, pl.ds(t * 16, 16)] = jnp.zeros((16,), jnp.float32)
  for q in range(_NPT // 128):
    pltpu.sync_copy(zrow.at[0], shf.at[pl.ds(si * _NPT + q * 128, 128)])
    pltpu.sync_copy(zrow.at[0], shs.at[pl.ds(si * _NPT + q * 128, 128)])
  plsc.subcore_barrier()

  @pl.loop(0, _NCHUNK)
  def _func_chunk(k):
    rowbase = wid * _ROWS_PW + k * _CHR
    pltpu.sync_copy(cf_h.at[pl.ds(rowbase, _CHR)], idxb)
    pltpu.sync_copy(ew_h.at[pl.ds(rowbase, _CHR)], updb)
    descs = [
        pltpu.async_copy(updb.at[j], shf.at[idxb.at[j]], sem, add=True)
        for j in range(_CHR)
    ]
    for d in descs:
      d.wait()

  @pl.loop(0, _NCHUNK)
  def _struct_chunk(k):
    rowbase = wid * _ROWS_PW + k * _CHR
    pltpu.sync_copy(cs_h.at[pl.ds(rowbase, _CHR)], idxb)
    descs = [
        pltpu.async_copy(onesb.at[0], shs.at[idxb.at[j]], sem, add=True)
        for j in range(_CHR)
    ]
    for d in descs:
      d.wait()

  plsc.subcore_barrier()
  pltpu.sync_copy(shf.at[pl.ds(si * _NPT, _NPT)],
                  out_h.at[ci, 0, pl.ds(si * _NPT, _NPT)])
  pltpu.sync_copy(shs.at[pl.ds(si * _NPT, _NPT)],
                  out_h.at[ci, 1, pl.ds(si * _NPT, _NPT)])


@functools.lru_cache
def _make_deg_kernel():
  scratch = [
      pltpu.VMEM_SHARED((_N,), jnp.float32),
      pltpu.VMEM_SHARED((_N,), jnp.float32),
      pltpu.VMEM((_CHR, 128), jnp.int32),
      pltpu.VMEM((_CHR, 128), jnp.float32),
      pltpu.VMEM((1, 128), jnp.float32),
      pltpu.VMEM((1, 128), jnp.float32),
      pltpu.SemaphoreType.DMA,
  ]
  return pl.kernel(
      _deg_body,
      out_type=jax.ShapeDtypeStruct((_NC, 2, _N), jnp.float32),
      mesh=_sc_mesh(),
      compiler_params=_SC_PARAMS,
      scratch_types=scratch,
  )


def _make_prop_body(K, has_w, nbuf, tbl_in_spmem):
  def body(*refs):
    if has_w:
      (r_h, c_h, w_h, tbl_h, out_h, *rest) = refs
    else:
      (r_h, c_h, tbl_h, out_h, *rest) = refs
    if tbl_in_spmem:
      (sht, sha, rb, cb, *more) = rest
    else:
      (sha, rb, cb, *more) = rest
      sht = tbl_h
    if has_w:
      (wb, *Gs) = more[:-2]
    else:
      Gs = more[:-2]
    sem, sem2 = more[-2], more[-1]
    G = tuple(Gs)
    ci = lax.axis_index("c")
    si = lax.axis_index("s")
    wid = si * _NC + ci
    if tbl_in_spmem:
      pltpu.sync_copy(tbl_h.at[pl.ds(si * _NPT, _NPT)],
                      sht.at[pl.ds(si * _NPT, _NPT)])
    for t in range(8):
      ei = lax.iota(jnp.int32, 16) + (t * 16)
      for kk in range(K):
        ki = jnp.full((16,), kk, jnp.int32)
        plsc.store_scatter(G[0], [ei, ki], jnp.zeros((16,), jnp.float32))
    for q in range(_NPT // 128):
      pltpu.sync_copy(G[0], sha.at[pl.ds(si * _NPT + q * 128, 128)])
    plsc.subcore_barrier()

    @pl.loop(0, _NCHUNK)
    def _chunk(k):
      rowbase = wid * _ROWS_PW + k * _CHR
      pltpu.sync_copy(r_h.at[pl.ds(rowbase, _CHR)], rb)
      pltpu.sync_copy(c_h.at[pl.ds(rowbase, _CHR)], cb)
      if has_w:
        pltpu.sync_copy(w_h.at[pl.ds(rowbase, _CHR)], wb)
      gds = [None] * _CHR
      sds = [None] * _CHR
      gds[0] = pltpu.async_copy(sht.at[rb.at[0]], G[0], sem)
      for j in range(_CHR):
        gds[j].wait()
        if nbuf > 1 and j >= nbuf - 1:
          sds[j - (nbuf - 1)].wait()
        if nbuf > 1 and j + 1 < _CHR:
          gds[j + 1] = pltpu.async_copy(sht.at[rb.at[j + 1]],
                                        G[(j + 1) % nbuf], sem)
        if has_w:
          for t in range(8):
            coef = wb[j, pl.ds(t * 16, 16)]
            ei = lax.iota(jnp.int32, 16) + (t * 16)
            for kk in range(_NUM_BANDS):
              ki = jnp.full((16,), kk, jnp.int32)
              g = plsc.load_gather(G[j % nbuf], [ei, ki])
              plsc.store_scatter(G[j % nbuf], [ei, ki], g * coef)
        sds[j] = pltpu.async_copy(G[j % nbuf], sha.at[cb.at[j]], sem2,
                                  add=True)
        if nbuf == 1:
          sds[j].wait()
          if j + 1 < _CHR:
            gds[j + 1] = pltpu.async_copy(sht.at[rb.at[j + 1]], G[0], sem)
      if nbuf > 1:
        for j in range(max(0, _CHR - (nbuf - 1)), _CHR):
          sds[j].wait()

    plsc.subcore_barrier()
    pltpu.sync_copy(sha.at[pl.ds(si * _NPT, _NPT)],
                    out_h.at[ci, pl.ds(si * _NPT, _NPT)])

  return body


@functools.lru_cache
def _make_prop_kernel(K, has_w, nbuf, tbl_in_spmem):
  scratch = []
  if tbl_in_spmem:
    scratch.append(pltpu.VMEM_SHARED((_N, K), jnp.float32))
  scratch += [
      pltpu.VMEM_SHARED((_N, K), jnp.float32),
      pltpu.VMEM((_CHR, 128), jnp.int32),
      pltpu.VMEM((_CHR, 128), jnp.int32),
  ]
  if has_w:
    scratch.append(pltpu.VMEM((_CHR, 128), jnp.float32))
  for _ in range(nbuf):
    scratch.append(pltpu.VMEM((128, K), jnp.float32))
  scratch += [pltpu.SemaphoreType.DMA, pltpu.SemaphoreType.DMA]
  return pl.kernel(
      _make_prop_body(K, has_w, nbuf, tbl_in_spmem),
      out_type=jax.ShapeDtypeStruct((_NC, _N, K), jnp.float32),
      mesh=_sc_mesh(),
      compiler_params=_SC_PARAMS,
      scratch_types=scratch,
  )


def _tc_prep(degp, x):
  d00 = degp[0, 0].reshape(_N, 1)
  d01 = degp[0, 1].reshape(_N, 1)
  d10 = degp[1, 0].reshape(_N, 1)
  d11 = degp[1, 1].reshape(_N, 1)
  BN = 4864
  grid = (_N // BN,)

  def body(a, b, c, d, x_, xd_ref, df_ref, ds_ref):
    df = lax.rsqrt(a[...] + c[...] + 1.0)
    ds_ = lax.rsqrt(b[...] + d[...] + 1.0)
    df_ref[...] = df
    ds_ref[...] = ds_
    xd_ref[...] = jnp.concatenate(
        [x_[...] * df, jnp.zeros((x_.shape[0], 3), jnp.float32)], axis=1)

  col = pl.BlockSpec((BN, 1), lambda i: (i, 0))
  row = pl.BlockSpec((BN, _NUM_BANDS), lambda i: (i, 0))
  row8 = pl.BlockSpec((BN, 8), lambda i: (i, 0))
  return pl.pallas_call(
      body,
      grid=grid,
      in_specs=[col, col, col, col, row],
      out_specs=(row8, col, col),
      out_shape=(jax.ShapeDtypeStruct((_N, 8), jnp.float32),
                 jax.ShapeDtypeStruct((_N, 1), jnp.float32),
                 jax.ShapeDtypeStruct((_N, 1), jnp.float32)),
  )(d00, d01, d10, d11, x)


def _tc_band(S0, S1, x, dinv_f, dinv_s, Aexp, Cf, W2bd):
  BN = 4864
  grid = (_N // BN,)

  def body(s0, s1, x_, df, dsr, ae, cf_, w2, td_ref, p_ref):
    dv = df[...]
    S = (s0[...][:, :_NUM_BANDS] + s1[...][:, :_NUM_BANDS]) * dv \
        + x_[...] * (dv * dv)
    H = jnp.maximum(
        jnp.dot(S, ae[...], preferred_element_type=jnp.float32) + cf_[...], 0.0)
    T = jnp.dot(H, w2[...], preferred_element_type=jnp.float32)
    dvs = dsr[...]
    td_ref[...] = jnp.concatenate(
        [T * dvs, jnp.zeros((T.shape[0], 6), jnp.float32)], axis=1)
    p_ref[...] = T * (dvs * dvs)

  row = pl.BlockSpec((BN, _NUM_BANDS), lambda i: (i, 0))
  row8 = pl.BlockSpec((BN, 8), lambda i: (i, 0))
  col = pl.BlockSpec((BN, 1), lambda i: (i, 0))
  row10 = pl.BlockSpec((BN, 10), lambda i: (i, 0))
  row16 = pl.BlockSpec((BN, 16), lambda i: (i, 0))
  full = lambda s: pl.BlockSpec(s, lambda i: (0,) * len(s))
  return pl.pallas_call(
      body,
      grid=grid,
      in_specs=[row8, row8, row, col, col,
                full((_NUM_BANDS, 160)), full((1, 160)), full((160, 10))],
      out_specs=(row16, row10),
      out_shape=(jax.ShapeDtypeStruct((_N, 16), jnp.float32),
                 jax.ShapeDtypeStruct((_N, 10), jnp.float32)),
  )(S0, S1, x, dinv_f, dinv_s, Aexp, Cf, W2bd)


def _tc_v(U0, U1, P, dinv_s, sc2, sh2):
  BN = 4864
  grid = (_N // BN,)

  def body(u0, u1, p_, dsr, sc, sh, v_ref):
    U = (u0[...][:, :10] + u1[...][:, :10]) * dsr[...] + p_[...]
    v_ref[...] = jnp.maximum(U * sc[...] + sh[...], 0.0)

  row10 = pl.BlockSpec((BN, 10), lambda i: (i, 0))
  row16 = pl.BlockSpec((BN, 16), lambda i: (i, 0))
  col = pl.BlockSpec((BN, 1), lambda i: (i, 0))
  full = lambda s: pl.BlockSpec(s, lambda i: (0,) * len(s))
  return pl.pallas_call(
      body,
      grid=grid,
      in_specs=[row16, row16, row10, col, full((1, 10)), full((1, 10))],
      out_specs=row10,
      out_shape=jax.ShapeDtypeStruct((_N, 10), jnp.float32),
  )(U0, U1, P, dinv_s, sc2, sh2)


def _tc_head(xc, W1pT, b1h, s3, be3, W2T, b2h, W3T, b3h):
  BB = 1024
  grid = (_BATCH // BB,)

  def body(x_, w1, b1_, s3_, be3_, w2, b2_, w3, b3_, o_ref):
    h = jnp.dot(x_[...], w1[...], preferred_element_type=jnp.float32) + b1_[...]
    h = jnp.maximum(h * s3_[...] + be3_[...], 0.0)
    h = jnp.maximum(
        jnp.dot(h, w2[...], preferred_element_type=jnp.float32) + b2_[...], 0.0)
    o_ref[...] = jnp.dot(h, w3[...], preferred_element_type=jnp.float32) + b3_[...]

  row = pl.BlockSpec((BB, 190), lambda i: (i, 0))
  full = lambda s: pl.BlockSpec(s, lambda i: (0,) * len(s))
  return pl.pallas_call(
      body,
      grid=grid,
      in_specs=[row, full((190, 128)), full((1, 128)), full((1, 128)),
                full((1, 128)), full((128, 32)), full((1, 32)),
                full((32, 2)), full((1, 2))],
      out_specs=pl.BlockSpec((BB, 2), lambda i: (i, 0)),
      out_shape=jax.ShapeDtypeStruct((_BATCH, 2), jnp.float32),
  )(xc, W1pT, b1h, s3, be3, W2T, b2h, W3T, b3h)


def kernel(x, edge_index_func, edge_weight_func, edge_index_struct, W1, b1, g1,
           be1, W2, b2, g2, be2, lin1_W, lin1_b, g3, be3, lin2_W, lin2_b,
           lin3_W, lin3_b):
  f32 = jnp.float32
  rf2 = edge_index_func[0].reshape(_E // 128, 128)
  cf2 = edge_index_func[1].reshape(_E // 128, 128)
  ew2 = edge_weight_func.reshape(_E // 128, 128)
  rs2 = edge_index_struct[0].reshape(_E // 128, 128)
  cs2 = edge_index_struct[1].reshape(_E // 128, 128)

  bn1s = g1 / jnp.sqrt(1.0 + _EPS)
  A = W1[:, 0, :] * bn1s
  C = (b1 * bn1s + be1).reshape(1, 160)
  band_of_col = jnp.repeat(jnp.arange(_NUM_BANDS), 32).reshape(1, 160)
  Aexp = jnp.where(jnp.arange(_NUM_BANDS)[:, None] == band_of_col,
                   jnp.tile(A.reshape(1, 160), (_NUM_BANDS, 1)), 0.0)
  col_band = jnp.repeat(jnp.arange(_NUM_BANDS), 2).reshape(1, 10)
  row_band = jnp.repeat(jnp.arange(_NUM_BANDS), 32).reshape(160, 1)
  W2stack = W2.reshape(160, 2)
  W2bd = jnp.where(row_band == col_band, W2stack[:, jnp.arange(10) % 2], 0.0)

  bn2s = (g2 / jnp.sqrt(1.0 + _EPS)).reshape(1, 10)
  sh2 = (b2 * (g2 / jnp.sqrt(1.0 + _EPS)) + be2).reshape(1, 10)

  n_i = jnp.arange(190) // 10
  r_i = jnp.arange(190) % 10
  perm = (r_i // 2) * (2 * _NUM_NODES) + n_i * 2 + (r_i % 2)
  W1pT = lin1_W[:, perm].T
  s3 = (g3 / jnp.sqrt(1.0 + _EPS)).reshape(1, 128)

  degp = _make_deg_kernel()(cf2, ew2, cs2)
  xd, dinv_f, dinv_s = _tc_prep(degp, x)
  Sp = _make_prop_kernel(8, True, 2, True)(rf2, cf2, ew2, xd)
  Td, P = _tc_band(Sp[0], Sp[1], x, dinv_f, dinv_s, Aexp, C, W2bd)
  Up = _make_prop_kernel(16, False, 2, False)(rs2, cs2, Td)
  v = _tc_v(Up[0], Up[1], P, dinv_s, bn2s, sh2)
  return _tc_head(v.reshape(_BATCH, 190), W1pT, lin1_b.reshape(1, 128), s3,
                  be3.reshape(1, 128), lin2_W.T, lin2_b.reshape(1, 32),
                  lin3_W.T, lin3_b.reshape(1, 2))


# speedup vs baseline: 1.0060x; 1.0060x over previous
import functools
import jax
import jax.numpy as jnp
from jax import lax
from jax.experimental import pallas as pl
from jax.experimental.pallas import tpu as pltpu
from jax.experimental.pallas import tpu_sc as plsc

_NUM_NODES = 19
_NUM_BANDS = 5
_BATCH = 4096
_N = _BATCH * _NUM_NODES
_E = 1245184
_EPS = 1e-5
_NC, _NS = 2, 16
_NW = _NC * _NS
_NPT = _N // _NS
_CH = 2048
_CHR = _CH // 128
_EPW = _E // _NW
_NCHUNK = _EPW // _CH
_ROWS_PW = _EPW // 128


def _sc_mesh():
  return plsc.VectorSubcoreMesh(core_axis_name="c", subcore_axis_name="s",
                                num_cores=_NC, num_subcores=_NS)


_SC_PARAMS = pltpu.CompilerParams(needs_layout_passes=False,
                                 use_tc_tiling_on_sc=False)


def _deg_body(cf_h, ew_h, cs_h, out_h, shf, shs, idxb, updb, onesb, zrow, sem):
  ci = lax.axis_index("c")
  si = lax.axis_index("s")
  wid = si * _NC + ci
  for t in range(8):
    onesb[0, pl.ds(t * 16, 16)] = jnp.full((16,), 1.0, jnp.float32)
    zrow[0, pl.ds(t * 16, 16)] = jnp.zeros((16,), jnp.float32)
  for q in range(_NPT // 128):
    pltpu.sync_copy(zrow.at[0], shf.at[pl.ds(si * _NPT + q * 128, 128)])
    pltpu.sync_copy(zrow.at[0], shs.at[pl.ds(si * _NPT + q * 128, 128)])
  plsc.subcore_barrier()

  @pl.loop(0, _NCHUNK)
  def _func_chunk(k):
    rowbase = wid * _ROWS_PW + k * _CHR
    pltpu.sync_copy(cf_h.at[pl.ds(rowbase, _CHR)], idxb)
    pltpu.sync_copy(ew_h.at[pl.ds(rowbase, _CHR)], updb)
    descs = [
        pltpu.async_copy(updb.at[j], shf.at[idxb.at[j]], sem, add=True)
        for j in range(_CHR)
    ]
    for d in descs:
      d.wait()

  @pl.loop(0, _NCHUNK)
  def _struct_chunk(k):
    rowbase = wid * _ROWS_PW + k * _CHR
    pltpu.sync_copy(cs_h.at[pl.ds(rowbase, _CHR)], idxb)
    descs = [
        pltpu.async_copy(onesb.at[0], shs.at[idxb.at[j]], sem, add=True)
        for j in range(_CHR)
    ]
    for d in descs:
      d.wait()

  plsc.subcore_barrier()
  pltpu.sync_copy(shf.at[pl.ds(si * _NPT, _NPT)],
                  out_h.at[ci, 0, pl.ds(si * _NPT, _NPT)])
  pltpu.sync_copy(shs.at[pl.ds(si * _NPT, _NPT)],
                  out_h.at[ci, 1, pl.ds(si * _NPT, _NPT)])


@functools.lru_cache
def _make_deg_kernel():
  scratch = [
      pltpu.VMEM_SHARED((_N,), jnp.float32),
      pltpu.VMEM_SHARED((_N,), jnp.float32),
      pltpu.VMEM((_CHR, 128), jnp.int32),
      pltpu.VMEM((_CHR, 128), jnp.float32),
      pltpu.VMEM((1, 128), jnp.float32),
      pltpu.VMEM((1, 128), jnp.float32),
      pltpu.SemaphoreType.DMA,
  ]
  return pl.kernel(
      _deg_body,
      out_type=jax.ShapeDtypeStruct((_NC, 2, _N), jnp.float32),
      mesh=_sc_mesh(),
      compiler_params=_SC_PARAMS,
      scratch_types=scratch,
  )


def _make_prop_body(K, has_w, nbuf, tbl_in_spmem):
  def body(*refs):
    if has_w:
      (r_h, c_h, w_h, tbl_h, out_h, *rest) = refs
    else:
      (r_h, c_h, tbl_h, out_h, *rest) = refs
    if tbl_in_spmem:
      (sht, sha, rb, cb, *more) = rest
    else:
      (sha, rb, cb, *more) = rest
      sht = tbl_h
    if has_w:
      (wb, *Gs) = more[:-2]
    else:
      Gs = more[:-2]
    sem, sem2 = more[-2], more[-1]
    G = tuple(Gs)
    ci = lax.axis_index("c")
    si = lax.axis_index("s")
    wid = si * _NC + ci
    if tbl_in_spmem:
      pltpu.sync_copy(tbl_h.at[pl.ds(si * _NPT, _NPT)],
                      sht.at[pl.ds(si * _NPT, _NPT)])
    for t in range(8):
      ei = lax.iota(jnp.int32, 16) + (t * 16)
      for kk in range(K):
        ki = jnp.full((16,), kk, jnp.int32)
        plsc.store_scatter(G[0], [ei, ki], jnp.zeros((16,), jnp.float32))
    for q in range(_NPT // 128):
      pltpu.sync_copy(G[0], sha.at[pl.ds(si * _NPT + q * 128, 128)])
    plsc.subcore_barrier()

    @pl.loop(0, _NCHUNK)
    def _chunk(k):
      rowbase = wid * _ROWS_PW + k * _CHR
      pltpu.sync_copy(r_h.at[pl.ds(rowbase, _CHR)], rb)
      pltpu.sync_copy(c_h.at[pl.ds(rowbase, _CHR)], cb)
      if has_w:
        pltpu.sync_copy(w_h.at[pl.ds(rowbase, _CHR)], wb)
      gds = [None] * _CHR
      sds = [None] * _CHR
      gds[0] = pltpu.async_copy(sht.at[rb.at[0]], G[0], sem)
      for j in range(_CHR):
        gds[j].wait()
        if nbuf > 1 and j >= nbuf - 1:
          sds[j - (nbuf - 1)].wait()
        if nbuf > 1 and j + 1 < _CHR:
          gds[j + 1] = pltpu.async_copy(sht.at[rb.at[j + 1]],
                                        G[(j + 1) % nbuf], sem)
        if has_w:
          for t in range(8):
            coef = wb[j, pl.ds(t * 16, 16)]
            ei = lax.iota(jnp.int32, 16) + (t * 16)
            for kk in range(_NUM_BANDS):
              ki = jnp.full((16,), kk, jnp.int32)
              g = plsc.load_gather(G[j % nbuf], [ei, ki])
              plsc.store_scatter(G[j % nbuf], [ei, ki], g * coef)
        sds[j] = pltpu.async_copy(G[j % nbuf], sha.at[cb.at[j]], sem2,
                                  add=True)
        if nbuf == 1:
          sds[j].wait()
          if j + 1 < _CHR:
            gds[j + 1] = pltpu.async_copy(sht.at[rb.at[j + 1]], G[0], sem)
      if nbuf > 1:
        for j in range(max(0, _CHR - (nbuf - 1)), _CHR):
          sds[j].wait()

    plsc.subcore_barrier()
    pltpu.sync_copy(sha.at[pl.ds(si * _NPT, _NPT)],
                    out_h.at[ci, pl.ds(si * _NPT, _NPT)])

  return body


@functools.lru_cache
def _make_prop_kernel(K, has_w, nbuf, tbl_in_spmem):
  scratch = []
  if tbl_in_spmem:
    scratch.append(pltpu.VMEM_SHARED((_N, K), jnp.float32))
  scratch += [
      pltpu.VMEM_SHARED((_N, K), jnp.float32),
      pltpu.VMEM((_CHR, 128), jnp.int32),
      pltpu.VMEM((_CHR, 128), jnp.int32),
  ]
  if has_w:
    scratch.append(pltpu.VMEM((_CHR, 128), jnp.float32))
  for _ in range(nbuf):
    scratch.append(pltpu.VMEM((128, K), jnp.float32))
  scratch += [pltpu.SemaphoreType.DMA, pltpu.SemaphoreType.DMA]
  return pl.kernel(
      _make_prop_body(K, has_w, nbuf, tbl_in_spmem),
      out_type=jax.ShapeDtypeStruct((_NC, _N, K), jnp.float32),
      mesh=_sc_mesh(),
      compiler_params=_SC_PARAMS,
      scratch_types=scratch,
  )


def _tc_prep(degp, x):
  d00 = degp[0, 0].reshape(_N, 1)
  d01 = degp[0, 1].reshape(_N, 1)
  d10 = degp[1, 0].reshape(_N, 1)
  d11 = degp[1, 1].reshape(_N, 1)
  BN = 4864
  grid = (_N // BN,)

  def body(a, b, c, d, x_, xd_ref, df_ref, ds_ref):
    df = lax.rsqrt(a[...] + c[...] + 1.0)
    ds_ = lax.rsqrt(b[...] + d[...] + 1.0)
    df_ref[...] = df
    ds_ref[...] = ds_
    xd_ref[...] = jnp.concatenate(
        [x_[...] * df, jnp.zeros((x_.shape[0], 3), jnp.float32)], axis=1)

  col = pl.BlockSpec((BN, 1), lambda i: (i, 0))
  row = pl.BlockSpec((BN, _NUM_BANDS), lambda i: (i, 0))
  row8 = pl.BlockSpec((BN, 8), lambda i: (i, 0))
  return pl.pallas_call(
      body,
      grid=grid,
      in_specs=[col, col, col, col, row],
      out_specs=(row8, col, col),
      out_shape=(jax.ShapeDtypeStruct((_N, 8), jnp.float32),
                 jax.ShapeDtypeStruct((_N, 1), jnp.float32),
                 jax.ShapeDtypeStruct((_N, 1), jnp.float32)),
  )(d00, d01, d10, d11, x)


def _tc_band(S0, S1, x, dinv_f, dinv_s, Aexp, Cf, W2bd):
  BN = 4864
  grid = (_N // BN,)

  def body(s0, s1, x_, df, dsr, ae, cf_, w2, td_ref, p_ref):
    dv = df[...]
    S = (s0[...][:, :_NUM_BANDS] + s1[...][:, :_NUM_BANDS]) * dv \
        + x_[...] * (dv * dv)
    H = jnp.maximum(
        jnp.dot(S, ae[...], preferred_element_type=jnp.float32) + cf_[...], 0.0)
    T = jnp.dot(H, w2[...], preferred_element_type=jnp.float32)
    dvs = dsr[...]
    td_ref[...] = jnp.concatenate(
        [T * dvs, jnp.zeros((T.shape[0], 6), jnp.float32)], axis=1)
    p_ref[...] = T * (dvs * dvs)

  row = pl.BlockSpec((BN, _NUM_BANDS), lambda i: (i, 0))
  row8 = pl.BlockSpec((BN, 8), lambda i: (i, 0))
  col = pl.BlockSpec((BN, 1), lambda i: (i, 0))
  row10 = pl.BlockSpec((BN, 10), lambda i: (i, 0))
  row16 = pl.BlockSpec((BN, 16), lambda i: (i, 0))
  full = lambda s: pl.BlockSpec(s, lambda i: (0,) * len(s))
  return pl.pallas_call(
      body,
      grid=grid,
      in_specs=[row8, row8, row, col, col,
                full((_NUM_BANDS, 160)), full((1, 160)), full((160, 10))],
      out_specs=(row16, row10),
      out_shape=(jax.ShapeDtypeStruct((_N, 16), jnp.float32),
                 jax.ShapeDtypeStruct((_N, 10), jnp.float32)),
  )(S0, S1, x, dinv_f, dinv_s, Aexp, Cf, W2bd)


def _tc_v(U0, U1, P, dinv_s, sc2, sh2):
  BN = 4864
  grid = (_N // BN,)

  def body(u0, u1, p_, dsr, sc, sh, v_ref):
    U = (u0[...][:, :10] + u1[...][:, :10]) * dsr[...] + p_[...]
    v_ref[...] = jnp.maximum(U * sc[...] + sh[...], 0.0)

  row10 = pl.BlockSpec((BN, 10), lambda i: (i, 0))
  row16 = pl.BlockSpec((BN, 16), lambda i: (i, 0))
  col = pl.BlockSpec((BN, 1), lambda i: (i, 0))
  full = lambda s: pl.BlockSpec(s, lambda i: (0,) * len(s))
  return pl.pallas_call(
      body,
      grid=grid,
      in_specs=[row16, row16, row10, col, full((1, 10)), full((1, 10))],
      out_specs=row10,
      out_shape=jax.ShapeDtypeStruct((_N, 10), jnp.float32),
  )(U0, U1, P, dinv_s, sc2, sh2)


def _tc_head(U0r, U1r, Pr, dsr, sc2, sh2, W1pT, b1h, s3, be3, W2T, b2h,
             W3T, b3h):
  BB = 1024
  grid = (_BATCH // BB,)

  def body(u0, u1, p_, ds_, sc, sh, w1, b1_, s3_, be3_, w2, b2_, w3, b3_,
           o_ref):
    u0v = u0[...]
    u1v = u1[...]
    dsv = ds_[...]
    parts = []
    for n in range(_NUM_NODES):
      un = (u0v[:, n * 16:n * 16 + 10] + u1v[:, n * 16:n * 16 + 10])
      parts.append(un * dsv[:, n:n + 1])
    U = jnp.concatenate(parts, axis=1) + p_[...]
    xc = jnp.maximum(U * sc[...] + sh[...], 0.0)
    h = jnp.dot(xc, w1[...], preferred_element_type=jnp.float32) + b1_[...]
    h = jnp.maximum(h * s3_[...] + be3_[...], 0.0)
    h = jnp.maximum(
        jnp.dot(h, w2[...], preferred_element_type=jnp.float32) + b2_[...], 0.0)
    o_ref[...] = jnp.dot(h, w3[...], preferred_element_type=jnp.float32) + b3_[...]

  row304 = pl.BlockSpec((BB, 304), lambda i: (i, 0))
  row190 = pl.BlockSpec((BB, 190), lambda i: (i, 0))
  row19 = pl.BlockSpec((BB, 19), lambda i: (i, 0))
  full = lambda s: pl.BlockSpec(s, lambda i: (0,) * len(s))
  return pl.pallas_call(
      body,
      grid=grid,
      in_specs=[row304, row304, row190, row19, full((1, 190)), full((1, 190)),
                full((190, 128)), full((1, 128)), full((1, 128)),
                full((1, 128)), full((128, 32)), full((1, 32)),
                full((32, 2)), full((1, 2))],
      out_specs=pl.BlockSpec((BB, 2), lambda i: (i, 0)),
      out_shape=jax.ShapeDtypeStruct((_BATCH, 2), jnp.float32),
  )(U0r, U1r, Pr, dsr, sc2, sh2, W1pT, b1h, s3, be3, W2T, b2h, W3T, b3h)


def kernel(x, edge_index_func, edge_weight_func, edge_index_struct, W1, b1, g1,
           be1, W2, b2, g2, be2, lin1_W, lin1_b, g3, be3, lin2_W, lin2_b,
           lin3_W, lin3_b):
  f32 = jnp.float32
  rf2 = edge_index_func[0].reshape(_E // 128, 128)
  cf2 = edge_index_func[1].reshape(_E // 128, 128)
  ew2 = edge_weight_func.reshape(_E // 128, 128)
  rs2 = edge_index_struct[0].reshape(_E // 128, 128)
  cs2 = edge_index_struct[1].reshape(_E // 128, 128)

  bn1s = g1 / jnp.sqrt(1.0 + _EPS)
  A = W1[:, 0, :] * bn1s
  C = (b1 * bn1s + be1).reshape(1, 160)
  band_of_col = jnp.repeat(jnp.arange(_NUM_BANDS), 32).reshape(1, 160)
  Aexp = jnp.where(jnp.arange(_NUM_BANDS)[:, None] == band_of_col,
                   jnp.tile(A.reshape(1, 160), (_NUM_BANDS, 1)), 0.0)
  col_band = jnp.repeat(jnp.arange(_NUM_BANDS), 2).reshape(1, 10)
  row_band = jnp.repeat(jnp.arange(_NUM_BANDS), 32).reshape(160, 1)
  W2stack = W2.reshape(160, 2)
  W2bd = jnp.where(row_band == col_band, W2stack[:, jnp.arange(10) % 2], 0.0)

  bn2s = jnp.tile((g2 / jnp.sqrt(1.0 + _EPS)).reshape(10), _NUM_NODES
                  ).reshape(1, 190)
  sh2 = jnp.tile((b2 * (g2 / jnp.sqrt(1.0 + _EPS)) + be2).reshape(10),
                 _NUM_NODES).reshape(1, 190)

  n_i = jnp.arange(190) // 10
  r_i = jnp.arange(190) % 10
  perm = (r_i // 2) * (2 * _NUM_NODES) + n_i * 2 + (r_i % 2)
  W1pT = lin1_W[:, perm].T
  s3 = (g3 / jnp.sqrt(1.0 + _EPS)).reshape(1, 128)

  degp = _make_deg_kernel()(cf2, ew2, cs2)
  xd, dinv_f, dinv_s = _tc_prep(degp, x)
  Sp = _make_prop_kernel(8, True, 2, True)(rf2, cf2, ew2, xd)
  Td, P = _tc_band(Sp[0], Sp[1], x, dinv_f, dinv_s, Aexp, C, W2bd)
  Up = _make_prop_kernel(16, False, 2, False)(rs2, cs2, Td)
  return _tc_head(Up[0].reshape(_BATCH, 304), Up[1].reshape(_BATCH, 304),
                  P.reshape(_BATCH, 190), dinv_s.reshape(_BATCH, _NUM_NODES),
                  bn2s, sh2, W1pT, lin1_b.reshape(1, 128), s3,
                  be3.reshape(1, 128), lin2_W.T, lin2_b.reshape(1, 32),
                  lin3_W.T, lin3_b.reshape(1, 2))
